# baseline (speedup 1.0000x reference)
import functools

import jax
import jax.numpy as jnp
from jax.experimental import pallas as pl

_B, _N1, _N2, _D = 4, 4096, 4096, 3
_DP = 8
_BLK = 512
_NB = _N1 // _BLK
_NH = _N2 // 2


def _chamfer_body(x1_ref, x2_ref, s1_ref, s2_ref):
    xb1 = x1_ref[0]
    xb2 = x2_ref[0]
    a2f = -2.0 * xb1
    sq1f = jnp.sum(xb1 * xb1, axis=0)
    sq2 = jnp.sum(xb2 * xb2, axis=0)
    rowsums = []
    cm = None
    for k in range(_NB):
        lo = k * _BLK
        a2 = a2f[:, lo:lo + _BLK]
        sq1b = sq1f[lo:lo + _BLK]
        cmks, vs = [], []
        for h in range(2):
            hl = h * _NH
            inner2 = jax.lax.dot_general(
                a2, xb2[:, hl:hl + _NH], (((0,), (0,)), ((), ())),
                preferred_element_type=jnp.float32)
            e = sq2[None, hl:hl + _NH] + inner2
            f = sq1b[:, None] + inner2
            vs.append(functools.reduce(
                jnp.minimum,
                [e[:, j * 128:(j + 1) * 128] for j in range(_NH // 128)]))
            cmks.append(jnp.min(f, axis=0))
        rowmin = jnp.min(
            functools.reduce(jnp.minimum, vs).T, axis=0) + sq1b
        rowsums.append(jnp.sum(jnp.maximum(rowmin, 0.0)))
        cmk = jnp.concatenate(cmks)
        cm = cmk if cm is None else jnp.minimum(cm, cmk)
    s1_ref[...] = sum(rowsums).reshape(1, 1, 1)
    s2_ref[...] = jnp.sum(
        jnp.maximum(cm + sq2, 0.0)).reshape(1, 1, 1)


def kernel(xyz1, xyz2):
    x1t = jnp.pad(jnp.moveaxis(xyz1, -1, -2), ((0, 0), (0, _DP - _D), (0, 0)))
    x2t = jnp.pad(jnp.moveaxis(xyz2, -1, -2), ((0, 0), (0, _DP - _D), (0, 0)))

    s1, s2 = pl.pallas_call(
        _chamfer_body,
        grid=(_B,),
        in_specs=[
            pl.BlockSpec((1, _DP, _N1), lambda i: (i, 0, 0)),
            pl.BlockSpec((1, _DP, _N2), lambda i: (i, 0, 0)),
        ],
        out_specs=[
            pl.BlockSpec((1, 1, 1), lambda i: (i, 0, 0)),
            pl.BlockSpec((1, 1, 1), lambda i: (i, 0, 0)),
        ],
        out_shape=[
            jax.ShapeDtypeStruct((_B, 1, 1), jnp.float32),
            jax.ShapeDtypeStruct((_B, 1, 1), jnp.float32),
        ],
    )(x1t, x2t)

    return jnp.sum(s1) / (_B * _N1), jnp.sum(s2) / (_B * _N2)
